# baseline (speedup 1.0000x reference)
import functools

import jax
import jax.numpy as jnp
from jax import lax
from jax.experimental import pallas as pl
from jax.experimental.pallas import tpu as pltpu
from jax.experimental.pallas import tpu_sc as plsc

_H = 128
_W = 128
_B = 2
_COLS = _W * _B
_LPT = 16
_NT = _COLS // _LPT
_NW = _H // 32
_BIG = 1 << 20


_HUGE = jnp.inf


def _best_kernel(radar_ref, mde_ref, enc_ref, rt, mt, et):
    C, H = radar_ref.shape
    unroll = 4
    rt[...] = radar_ref[...].T
    mde_in = mde_ref[...]
    mt[...] = jnp.where(mde_in != 0.0, mde_in, _HUGE).T
    mdem = mt[...]
    has_mde = jnp.any(mdem != _HUGE, axis=0, keepdims=True)
    posi = lax.broadcasted_iota(jnp.int32, (H, C), 0)

    def step(i, carry):
        for k in range(unroll):
            y = i * unroll + k
            d_r = rt[pl.ds(y, 1), :]
            diffs = jnp.abs(mdem - d_r)
            m = jnp.min(diffs, axis=0, keepdims=True)
            bidx = jnp.min(
                jnp.where(diffs == m, posi, H), axis=0, keepdims=True
            )
            best = jnp.where(has_mde, bidx, y)
            et[pl.ds(y, 1), :] = jnp.where(d_r != 0.0, best, -1)
        return carry

    lax.fori_loop(0, H // unroll, step, 0)
    enc_ref[...] = et[...].T


def _lsb_exp(t):
    f = t.astype(jnp.float32)
    return ((lax.bitcast_convert_type(f, jnp.int32) >> 23) & 0xFF) - 127


def _sc_resolve_kernel(enc_hbm, vals_hbm, out_hbm, enc_v, vals_v, occ_t):
    wid = lax.axis_index("s") * 2 + lax.axis_index("c")

    @pl.when(wid < _NT)
    def _():
        base = wid * _LPT
        pltpu.sync_copy(enc_hbm.at[pl.ds(base, _LPT)], enc_v)
        pltpu.sync_copy(vals_hbm.at[pl.ds(base, _LPT)], vals_v)
        lanes = lax.broadcasted_iota(jnp.int32, (16,), 0)
        zero16 = jnp.zeros((16,), jnp.int32)
        ones = jnp.full((16,), 1, jnp.int32)
        full = jnp.full((16,), -1, jnp.int32)
        fzero = jnp.zeros((16,), jnp.float32)
        for c in range(_LPT):
            for k in range(_H // 16):
                occ_t[c, pl.ds(16 * k, 16)] = fzero

        def lsr(x, k):
            return lax.shift_right_logical(x, jnp.int32(k))

        def step(y, fw):
            y16 = zero16 + y
            b = plsc.load_gather(enc_v, [lanes, y16])
            vals = plsc.load_gather(vals_v, [lanes, y16])
            wb = b >> 5
            rb = b & 31
            hb = ones << rb
            himask = 0 - hb
            lomask = hb | (hb - 1)
            vu, wu = ones, jnp.full((16,), _NW, jnp.int32)
            for i in range(_NW - 1, -1, -1):
                sel = jnp.where(wb < i, full, jnp.where(wb == i, himask, 0))
                mi = fw[i] & sel
                nz = mi != 0
                vu = jnp.where(nz, mi, vu)
                wu = jnp.where(nz, i, wu)
            fu = wu * 32 + _lsb_exp(vu & (0 - vu))
            vd, wd = ones, jnp.full((16,), -_NW, jnp.int32)
            for i in range(_NW):
                sel = jnp.where(wb > i, full, jnp.where(wb == i, lomask, 0))
                mi = fw[i] & sel
                nz = mi != 0
                vd = jnp.where(nz, mi, vd)
                wd = jnp.where(nz, i, wd)
            s = vd | lsr(vd, 1)
            s = s | lsr(s, 2)
            s = s | lsr(s, 4)
            s = s | lsr(s, 8)
            s = s | lsr(s, 16)
            fd = wd * 32 + _lsb_exp(s ^ lsr(s, 1))
            du = jnp.where(fu < _H, fu - b, _BIG)
            dd = jnp.where(fd >= 0, b - fd, _BIG)
            final = jnp.clip(jnp.where(du <= dd, fu, fd), 0, _H - 1)
            write = b >= 0
            plsc.store_scatter(occ_t, [lanes, final], vals, mask=write)
            wf = final >> 5
            clearbit = jnp.where(write, ones << (final & 31), zero16)
            return tuple(
                jnp.where(wf == i, fw[i] & ~clearbit, fw[i])
                for i in range(_NW)
            )

        lax.fori_loop(0, _H, step, (full,) * _NW)
        pltpu.sync_copy(occ_t, out_hbm.at[pl.ds(base, _LPT)])


@functools.lru_cache(maxsize=None)
def _sc_resolve():
    return pl.kernel(
        _sc_resolve_kernel,
        out_type=jax.ShapeDtypeStruct((_COLS, _H), jnp.float32),
        mesh=plsc.VectorSubcoreMesh(core_axis_name="c", subcore_axis_name="s"),
        compiler_params=pltpu.CompilerParams(needs_layout_passes=False),
        scratch_types=[
            pltpu.VMEM((_LPT, _H), jnp.int32),
            pltpu.VMEM((_LPT, _H), jnp.float32),
            pltpu.VMEM((_LPT, _H), jnp.float32),
        ],
    )


def kernel(radar_patches, mde_out_patches):
    W, B, C, H, _ = radar_patches.shape
    radar_wbh = radar_patches[:, :, 0, :, 0]
    radar_cols = radar_wbh.reshape(W * B, H)
    mde_cols = mde_out_patches[:, :, 0, :, 0].reshape(W * B, H)

    enc = pl.pallas_call(
        _best_kernel,
        out_shape=jax.ShapeDtypeStruct((W * B, H), jnp.int32),
        scratch_shapes=[
            pltpu.VMEM((H, W * B), jnp.float32),
            pltpu.VMEM((H, W * B), jnp.float32),
            pltpu.VMEM((H, W * B), jnp.int32),
        ],
    )(radar_cols, mde_cols)

    occ = _sc_resolve()(enc, radar_cols)
    cols_t = jnp.transpose(occ.reshape(W, B, H), (1, 2, 0))
    if C == 1:
        return cols_t[:, None, :, :]
    radar_gt = jnp.zeros((B, C, H, W), dtype=jnp.float32)
    return radar_gt.at[:, 0, :, :].set(cols_t)


# speedup vs baseline: 1.0106x; 1.0106x over previous
import functools

import jax
import jax.numpy as jnp
from jax import lax
from jax.experimental import pallas as pl
from jax.experimental.pallas import tpu as pltpu
from jax.experimental.pallas import tpu_sc as plsc

_H = 128
_W = 128
_B = 2
_COLS = _W * _B
_LPT = 16
_NT = _COLS // _LPT
_NW = _H // 32
_BIG = 1 << 20


_HUGE = jnp.inf


def _best_kernel(radar_ref, mde_ref, enc_ref, rt, mt, et):
    C, H = radar_ref.shape
    unroll = 8
    rt[...] = radar_ref[...].T
    mde_in = mde_ref[...]
    mt[...] = jnp.where(mde_in != 0.0, mde_in, _HUGE).T
    mdem = mt[...]
    has_mde = jnp.any(mdem != _HUGE, axis=0, keepdims=True)
    posi = lax.broadcasted_iota(jnp.int32, (H, C), 0)

    def step(i, carry):
        for k in range(unroll):
            y = i * unroll + k
            d_r = rt[pl.ds(y, 1), :]
            diffs = jnp.abs(mdem - d_r)
            m = jnp.min(diffs, axis=0, keepdims=True)
            bidx = jnp.min(
                jnp.where(diffs == m, posi, H), axis=0, keepdims=True
            )
            best = jnp.where(has_mde, bidx, y)
            et[pl.ds(y, 1), :] = jnp.where(d_r != 0.0, best, -1)
        return carry

    lax.fori_loop(0, H // unroll, step, 0)
    enc_ref[...] = et[...].T


def _lsb_exp(t):
    f = t.astype(jnp.float32)
    return ((lax.bitcast_convert_type(f, jnp.int32) >> 23) & 0xFF) - 127


def _sc_resolve_kernel(enc_hbm, vals_hbm, out_hbm, enc_v, vals_v, occ_t):
    wid = lax.axis_index("s") * 2 + lax.axis_index("c")

    @pl.when(wid < _NT)
    def _():
        base = wid * _LPT
        pltpu.sync_copy(enc_hbm.at[pl.ds(base, _LPT)], enc_v)
        pltpu.sync_copy(vals_hbm.at[pl.ds(base, _LPT)], vals_v)
        lanes = lax.broadcasted_iota(jnp.int32, (16,), 0)
        zero16 = jnp.zeros((16,), jnp.int32)
        ones = jnp.full((16,), 1, jnp.int32)
        full = jnp.full((16,), -1, jnp.int32)
        fzero = jnp.zeros((16,), jnp.float32)
        for c in range(_LPT):
            for k in range(_H // 16):
                occ_t[c, pl.ds(16 * k, 16)] = fzero

        def lsr(x, k):
            return lax.shift_right_logical(x, jnp.int32(k))

        def step(y, fw):
            y16 = zero16 + y
            b = plsc.load_gather(enc_v, [lanes, y16])
            vals = plsc.load_gather(vals_v, [lanes, y16])
            wb = b >> 5
            rb = b & 31
            hb = ones << rb
            himask = 0 - hb
            lomask = hb | (hb - 1)
            vu, wu = ones, jnp.full((16,), _NW, jnp.int32)
            for i in range(_NW - 1, -1, -1):
                sel = jnp.where(wb < i, full, jnp.where(wb == i, himask, 0))
                mi = fw[i] & sel
                nz = mi != 0
                vu = jnp.where(nz, mi, vu)
                wu = jnp.where(nz, i, wu)
            fu = wu * 32 + _lsb_exp(vu & (0 - vu))
            vd, wd = ones, jnp.full((16,), -_NW, jnp.int32)
            for i in range(_NW):
                sel = jnp.where(wb > i, full, jnp.where(wb == i, lomask, 0))
                mi = fw[i] & sel
                nz = mi != 0
                vd = jnp.where(nz, mi, vd)
                wd = jnp.where(nz, i, wd)
            s = vd | lsr(vd, 1)
            s = s | lsr(s, 2)
            s = s | lsr(s, 4)
            s = s | lsr(s, 8)
            s = s | lsr(s, 16)
            fd = wd * 32 + _lsb_exp(s ^ lsr(s, 1))
            du = jnp.where(fu < _H, fu - b, _BIG)
            dd = jnp.where(fd >= 0, b - fd, _BIG)
            final = jnp.clip(jnp.where(du <= dd, fu, fd), 0, _H - 1)
            write = b >= 0
            plsc.store_scatter(occ_t, [lanes, final], vals, mask=write)
            wf = final >> 5
            clearbit = jnp.where(write, ones << (final & 31), zero16)
            return tuple(
                jnp.where(wf == i, fw[i] & ~clearbit, fw[i])
                for i in range(_NW)
            )

        lax.fori_loop(0, _H, step, (full,) * _NW)
        pltpu.sync_copy(occ_t, out_hbm.at[pl.ds(base, _LPT)])


@functools.lru_cache(maxsize=None)
def _sc_resolve():
    return pl.kernel(
        _sc_resolve_kernel,
        out_type=jax.ShapeDtypeStruct((_COLS, _H), jnp.float32),
        mesh=plsc.VectorSubcoreMesh(core_axis_name="c", subcore_axis_name="s"),
        compiler_params=pltpu.CompilerParams(needs_layout_passes=False),
        scratch_types=[
            pltpu.VMEM((_LPT, _H), jnp.int32),
            pltpu.VMEM((_LPT, _H), jnp.float32),
            pltpu.VMEM((_LPT, _H), jnp.float32),
        ],
    )


def kernel(radar_patches, mde_out_patches):
    W, B, C, H, _ = radar_patches.shape
    radar_wbh = radar_patches[:, :, 0, :, 0]
    radar_cols = radar_wbh.reshape(W * B, H)
    mde_cols = mde_out_patches[:, :, 0, :, 0].reshape(W * B, H)

    enc = pl.pallas_call(
        _best_kernel,
        out_shape=jax.ShapeDtypeStruct((W * B, H), jnp.int32),
        scratch_shapes=[
            pltpu.VMEM((H, W * B), jnp.float32),
            pltpu.VMEM((H, W * B), jnp.float32),
            pltpu.VMEM((H, W * B), jnp.int32),
        ],
    )(radar_cols, mde_cols)

    occ = _sc_resolve()(enc, radar_cols)
    cols_t = jnp.transpose(occ.reshape(W, B, H), (1, 2, 0))
    if C == 1:
        return cols_t[:, None, :, :]
    radar_gt = jnp.zeros((B, C, H, W), dtype=jnp.float32)
    return radar_gt.at[:, 0, :, :].set(cols_t)
